# Initial kernel scaffold; baseline (speedup 1.0000x reference)
import jax
import jax.numpy as jnp
from jax.experimental import pallas as pl


def kernel(sent_embs, frag_lengths, beg_seq_param, end_seq_param, W_enc):
    raise NotImplementedError("write your pallas kernel here")


# speedup vs baseline: 16.1399x; 16.1399x over previous
import jax
import jax.numpy as jnp
from jax.experimental import pallas as pl
from jax.experimental.pallas import tpu as pltpu


_ROWS_PER_STEP = 2048


def _seg_pool_body(cu_beg_ref, cu_end_ref, be_ref, len_ref, x_ref, w_ref,
                   o_ref, *, total, rows_per_step):
    step = pl.program_id(0)
    nsteps = pl.num_programs(0)
    r = rows_per_step
    row0 = step * r
    rows = row0 + jax.lax.broadcasted_iota(jnp.int32, (1, r), 1)
    onehot = ((rows >= cu_beg_ref[:, :]) & (rows < cu_end_ref[:, :]))
    onehot = onehot.astype(jnp.float32)
    x = x_ref[:, :]
    if total % r != 0:
        rows_col = row0 + jax.lax.broadcasted_iota(jnp.int32, (r, 1), 0)
        x = jnp.where(rows_col < total, x, 0.0)
    partial = jnp.dot(onehot, x, preferred_element_type=jnp.float32)

    @pl.when(step == 0)
    def _():
        o_ref[:, :] = jnp.zeros_like(o_ref)

    o_ref[:, :] += partial

    @pl.when(step == nsteps - 1)
    def _():
        pooled = (o_ref[:, :] + be_ref[:, :]) / len_ref[:, :]
        o_ref[:, :] = jnp.dot(pooled, w_ref[:, :],
                              preferred_element_type=jnp.float32)


def kernel(sent_embs, frag_lengths, beg_seq_param, end_seq_param, W_enc):
    total, d = sent_embs.shape
    b = frag_lengths.shape[0]
    r = _ROWS_PER_STEP
    nsteps = (total + r - 1) // r

    cu = jnp.concatenate([jnp.zeros((1,), dtype=frag_lengths.dtype),
                          jnp.cumsum(frag_lengths)])
    cu_beg = cu[:b].reshape(b, 1)
    cu_end = cu[1:].reshape(b, 1)
    be = (beg_seq_param + end_seq_param).reshape(1, d)
    len2 = (frag_lengths + 2).astype(jnp.float32).reshape(b, 1)

    import functools
    body = functools.partial(_seg_pool_body, total=total, rows_per_step=r)

    return pl.pallas_call(
        body,
        grid=(nsteps,),
        in_specs=[
            pl.BlockSpec((b, 1), lambda i: (0, 0)),
            pl.BlockSpec((b, 1), lambda i: (0, 0)),
            pl.BlockSpec((1, d), lambda i: (0, 0)),
            pl.BlockSpec((b, 1), lambda i: (0, 0)),
            pl.BlockSpec((r, d), lambda i: (i, 0)),
            pl.BlockSpec((d, d), lambda i: (0, 0)),
        ],
        out_specs=pl.BlockSpec((b, d), lambda i: (0, 0)),
        out_shape=jax.ShapeDtypeStruct((b, d), jnp.float32),
    )(cu_beg, cu_end, be, len2, sent_embs, W_enc)
